# baseline (speedup 1.0000x reference)
import functools

import jax
import jax.numpy as jnp
from jax import lax
from jax.experimental import pallas as pl

_K = 16
_G = 4
_BB = 4


def _level(x_ref, w1_ref, b1_ref, w2_ref, y_ref, m_ref, n, bb):
    w1 = w1_ref[...]
    b1 = b1_ref[...]
    w2 = w2_ref[...]
    rows = []
    for b in range(bb):
        xi = x_ref[0, b]
        h = jnp.dot(w1, xi, preferred_element_type=jnp.float32)
        h = h + b1
        h = 0.5 * h * (1.0 + lax.erf(h * 0.7071067811865476))
        rows.append(jnp.dot(w2, h, preferred_element_type=jnp.float32))
    scores = jnp.concatenate(rows, axis=0)

    iota = lax.broadcasted_iota(jnp.int32, (bb, n), 1)

    def step(_, taken):
        cur = jnp.where(taken, -jnp.inf, scores)
        m = jnp.max(cur, axis=1, keepdims=True)
        idx = jnp.min(jnp.where(cur == m, iota, n), axis=1, keepdims=True)
        return taken | (iota == idx)

    taken = lax.fori_loop(0, _K, step, jnp.zeros((bb, n), jnp.bool_),
                          unroll=True)
    m_ref[0] = taken
    mf = taken.astype(jnp.float32)
    y_ref[0] = x_ref[0] * mf[:, None, :]


def _body(x5_ref, x4_ref, x3_ref,
          w15_ref, b15_ref, w25_ref,
          w14_ref, b14_ref, w24_ref,
          w13_ref, b13_ref, w23_ref,
          y5_ref, y4_ref, y3_ref, m5_ref, m4_ref, m3_ref, *, ns, bb):
    s = pl.program_id(0)

    @pl.when(s < _G)
    def _():
        _level(x5_ref, w15_ref, b15_ref, w25_ref, y5_ref, m5_ref, ns[2], bb)

    @pl.when((s >= _G) & (s < 2 * _G))
    def _():
        _level(x4_ref, w14_ref, b14_ref, w24_ref, y4_ref, m4_ref, ns[1], bb)

    @pl.when(s >= 2 * _G)
    def _():
        _level(x3_ref, w13_ref, b13_ref, w23_ref, y3_ref, m3_ref, ns[0], bb)


def kernel(P3, P4, P5, budget, W1_P3, b1_P3, W2_P3, b2_P3,
           W1_P4, b1_P4, W2_P4, b2_P4, W1_P5, b1_P5, W2_P5, b2_P5):
    b, c = P3.shape[:2]
    shapes = [P3.shape, P4.shape, P5.shape]
    ns = tuple(s[2] * s[3] for s in shapes)
    bb = _BB
    xs = [x.reshape(_G, bb, c, n) for x, n in zip((P3, P4, P5), ns)]

    def phase_idx(phase):
        def f(s):
            return jnp.clip(s - phase * _G, 0, _G - 1)
        return f

    def xspec(n, phase):
        f = phase_idx(phase)
        return pl.BlockSpec((1, bb, c, n), lambda s: (f(s), 0, 0, 0))

    def mspec(n, phase):
        f = phase_idx(phase)
        return pl.BlockSpec((1, bb, n), lambda s: (f(s), 0, 0))

    def wspecs():
        return [
            pl.BlockSpec((64, 128), lambda s: (0, 0)),
            pl.BlockSpec((64, 1), lambda s: (0, 0)),
            pl.BlockSpec((1, 64), lambda s: (0, 0)),
        ]

    body = functools.partial(_body, ns=ns, bb=bb)
    outs = pl.pallas_call(
        body,
        grid=(3 * _G,),
        in_specs=[xspec(ns[2], 0), xspec(ns[1], 1), xspec(ns[0], 2)]
                 + wspecs() * 3,
        out_specs=[xspec(ns[2], 0), xspec(ns[1], 1), xspec(ns[0], 2),
                   mspec(ns[2], 0), mspec(ns[1], 1), mspec(ns[0], 2)],
        out_shape=[jax.ShapeDtypeStruct((_G, bb, c, ns[2]), jnp.float32),
                   jax.ShapeDtypeStruct((_G, bb, c, ns[1]), jnp.float32),
                   jax.ShapeDtypeStruct((_G, bb, c, ns[0]), jnp.float32),
                   jax.ShapeDtypeStruct((_G, bb, ns[2]), jnp.bool_),
                   jax.ShapeDtypeStruct((_G, bb, ns[1]), jnp.bool_),
                   jax.ShapeDtypeStruct((_G, bb, ns[0]), jnp.bool_)],
    )(xs[2], xs[1], xs[0],
      W1_P5, b1_P5.reshape(64, 1), W2_P5,
      W1_P4, b1_P4.reshape(64, 1), W2_P4,
      W1_P3, b1_P3.reshape(64, 1), W2_P3)
    y5, y4, y3 = outs[0], outs[1], outs[2]
    m5, m4, m3 = outs[3], outs[4], outs[5]
    k = jnp.array([_K], dtype=jnp.int32)
    return (y3.reshape(shapes[0]), y4.reshape(shapes[1]),
            y5.reshape(shapes[2]), m3.reshape(b, ns[0]),
            m4.reshape(b, ns[1]), m5.reshape(b, ns[2]), k, k, k)


# speedup vs baseline: 1.2582x; 1.2582x over previous
import functools

import jax
import jax.numpy as jnp
from jax import lax
from jax.experimental import pallas as pl

_K = 16
_G = 4


def _level(x_ref, w1_ref, b1_ref, w2_ref, y_ref, m_ref, n, bb):
    w1 = w1_ref[...]
    b1 = b1_ref[...]
    w2 = w2_ref[...]
    rows = []
    for b in range(bb):
        xi = x_ref[0, b]
        h = jnp.dot(w1, xi, preferred_element_type=jnp.float32)
        h = h + b1
        h = 0.5 * h * (1.0 + lax.erf(h * 0.7071067811865476))
        rows.append(jnp.dot(w2, h, preferred_element_type=jnp.float32))
    scores = jnp.concatenate(rows, axis=0)

    iota = lax.broadcasted_iota(jnp.int32, (bb, n), 1)

    def step(_, taken):
        cur = jnp.where(taken, -jnp.inf, scores)
        m = jnp.max(cur, axis=1, keepdims=True)
        idx = jnp.min(jnp.where(cur == m, iota, n), axis=1, keepdims=True)
        return taken | (iota == idx)

    taken = lax.fori_loop(0, _K, step, jnp.zeros((bb, n), jnp.bool_),
                          unroll=True)
    m_ref[0] = taken
    mf = taken.astype(jnp.float32)
    y_ref[0] = x_ref[0] * mf[:, None, :]


def _body(x3_ref, x4_ref, x5_ref,
          w13_ref, b13_ref, w23_ref,
          w14_ref, b14_ref, w24_ref,
          w15_ref, b15_ref, w25_ref,
          y3_ref, y4_ref, y5_ref, m3_ref, m4_ref, m5_ref, *, ns, bb):
    _level(x3_ref, w13_ref, b13_ref, w23_ref, y3_ref, m3_ref, ns[0], bb)
    _level(x4_ref, w14_ref, b14_ref, w24_ref, y4_ref, m4_ref, ns[1], bb)
    _level(x5_ref, w15_ref, b15_ref, w25_ref, y5_ref, m5_ref, ns[2], bb)


def kernel(P3, P4, P5, budget, W1_P3, b1_P3, W2_P3, b2_P3,
           W1_P4, b1_P4, W2_P4, b2_P4, W1_P5, b1_P5, W2_P5, b2_P5):
    b, c = P3.shape[:2]
    shapes = [P3.shape, P4.shape, P5.shape]
    ns = tuple(s[2] * s[3] for s in shapes)
    bb = b // _G
    xs = [x.reshape(_G, bb, c, n) for x, n in zip((P3, P4, P5), ns)]

    def xspec(n):
        return pl.BlockSpec((1, bb, c, n), lambda g: (g, 0, 0, 0))

    def mspec(n):
        return pl.BlockSpec((1, bb, n), lambda g: (g, 0, 0))

    wspecs = [
        pl.BlockSpec((64, 128), lambda g: (0, 0)),
        pl.BlockSpec((64, 1), lambda g: (0, 0)),
        pl.BlockSpec((1, 64), lambda g: (0, 0)),
    ]
    body = functools.partial(_body, ns=ns, bb=bb)
    outs = pl.pallas_call(
        body,
        grid=(_G,),
        in_specs=[xspec(ns[0]), xspec(ns[1]), xspec(ns[2])] + wspecs * 3,
        out_specs=[xspec(ns[0]), xspec(ns[1]), xspec(ns[2]),
                   mspec(ns[0]), mspec(ns[1]), mspec(ns[2])],
        out_shape=[jax.ShapeDtypeStruct((_G, bb, c, n), jnp.float32)
                   for n in ns] +
                  [jax.ShapeDtypeStruct((_G, bb, n), jnp.bool_) for n in ns],
    )(xs[0], xs[1], xs[2],
      W1_P3, b1_P3.reshape(64, 1), W2_P3,
      W1_P4, b1_P4.reshape(64, 1), W2_P4,
      W1_P5, b1_P5.reshape(64, 1), W2_P5)
    ys = [y.reshape(s) for y, s in zip(outs[:3], shapes)]
    ms = [m.reshape(b, n) for m, n in zip(outs[3:], ns)]
    k = jnp.array([_K], dtype=jnp.int32)
    return (ys[0], ys[1], ys[2], ms[0], ms[1], ms[2], k, k, k)
